# Initial kernel scaffold; baseline (speedup 1.0000x reference)
import jax
import jax.numpy as jnp
from jax.experimental import pallas as pl


def kernel(x, v, edge_index, edge_w, Wq0, Wk0, Wv0, Ws0, Ra0, Rb0, Wq1, Wk1, Wv1, Ws1, Ra1, Rb1, Wq2, Wk2, Wv2, Ws2, Ra2, Rb2, scale0, bias0, scale1, bias1):
    raise NotImplementedError("write your pallas kernel here")


# speedup vs baseline: 1.1041x; 1.1041x over previous
import jax
import jax.numpy as jnp
from jax.experimental import pallas as pl
from jax.experimental.pallas import tpu as pltpu


def _inv_kernel(h_ref, o_ref):
    h = h_ref[...]
    o_ref[...] = jnp.sqrt(jnp.sum(h * h, axis=-1) + 1e-12)


def _node_inv(h):
    n, m, _ = h.shape
    blk = 1000
    return pl.pallas_call(
        _inv_kernel,
        out_shape=jax.ShapeDtypeStruct((n, m), h.dtype),
        grid=(n // blk,),
        in_specs=[pl.BlockSpec((blk, m, 3), lambda i: (i, 0, 0))],
        out_specs=pl.BlockSpec((blk, m), lambda i: (i, 0)),
    )(h)


def _layer(h, x, src, dst, ew, Wq, Wk, Wv, Ws, Ra, Rb):
    n = h.shape[0]
    c = Wq.shape[1]
    rel = x[dst] - x[src]
    r = jnp.sqrt(jnp.sum(rel * rel, axis=-1, keepdims=True) + 1e-12)
    rel_dir = rel / r
    radial = jax.nn.relu(jnp.concatenate([r, ew], axis=-1) @ Ra) @ Rb
    inv = _node_inv(h)
    q = inv @ Wq
    k = inv @ Wk
    logits = jnp.sum(q[dst] * k[src], axis=-1) / jnp.sqrt(float(c))
    el = jnp.exp(logits - jnp.max(logits))
    den = jnp.maximum(jax.ops.segment_sum(el, dst, num_segments=n), 1e-38)
    vmsg = jnp.einsum('emd,mc->ecd', h[src], Wv) + radial[:, :, None] * rel_dir[:, None, :]
    num = jax.ops.segment_sum(el[:, None, None] * vmsg, dst, num_segments=n)
    agg = num / den[:, None, None]
    return agg + jnp.einsum('nmd,mc->ncd', h, Ws)


def _norm_bias(h, scale, bias):
    nrm = jnp.sqrt(jnp.sum(h * h, axis=-1) + 1e-12)
    phase = h / nrm[..., None]
    return phase * jax.nn.relu(nrm * scale + bias)[..., None]


def kernel(x, v, edge_index, edge_w, Wq0, Wk0, Wv0, Ws0, Ra0, Rb0, Wq1, Wk1, Wv1, Ws1, Ra1, Rb1, Wq2, Wk2, Wv2, Ws2, Ra2, Rb2, scale0, bias0, scale1, bias1):
    src = edge_index[0]
    dst = edge_index[1]
    h = v
    h = _layer(h, x, src, dst, edge_w, Wq0, Wk0, Wv0, Ws0, Ra0, Rb0)
    h = _norm_bias(h, scale0, bias0)
    h = _layer(h, x, src, dst, edge_w, Wq1, Wk1, Wv1, Ws1, Ra1, Rb1)
    h = _norm_bias(h, scale1, bias1)
    h = _layer(h, x, src, dst, edge_w, Wq2, Wk2, Wv2, Ws2, Ra2, Rb2)
    return h


# speedup vs baseline: 51.2128x; 46.3859x over previous
import functools
import math

import jax
import jax.numpy as jnp
from jax import lax
from jax.experimental import pallas as pl
from jax.experimental.pallas import tpu as pltpu
from jax.experimental.pallas import tpu_sc as plsc

N = 50000
E = 800000
EPAD = 819200
NB = 64
BW = 782
NPAD = NB * BW
CAP = 16384
BATCH = 128
ACC_W = 97
ACC_BLK = BW * ACC_W + 2
CH = 8192
DEAD = -65536
M16 = -65536

_mesh = plsc.VectorSubcoreMesh(core_axis_name="c", subcore_axis_name="s")


def _bf16r(x):
    b = lax.bitcast_convert_type(x, jnp.int32)
    t = b + jnp.int32(0x7FFF) + jnp.bitwise_and(lax.shift_right_logical(b, 16), 1)
    return lax.bitcast_convert_type(jnp.bitwise_and(t, M16), jnp.float32)


@functools.partial(
    pl.kernel,
    out_type=(
        jax.ShapeDtypeStruct((NB * CAP,), jnp.int32),
        jax.ShapeDtypeStruct((NB * CAP,), jnp.int32),
        jax.ShapeDtypeStruct((1024,), jnp.int32),
    ),
    mesh=_mesh,
    compiler_params=pltpu.CompilerParams(needs_layout_passes=False, use_tc_tiling_on_sc=False),
    scratch_types=[
        pltpu.VMEM((CH,), jnp.int32),
        pltpu.VMEM((CH,), jnp.int32),
        pltpu.VMEM((CAP,), jnp.int32),
        pltpu.VMEM((CAP,), jnp.int32),
        pltpu.VMEM((CAP,), jnp.int32),
        pltpu.VMEM((CAP,), jnp.int32),
        pltpu.VMEM((16,), jnp.int32),
    ],
)
def _partition(srcs_in, aux_in, srcs_out, aux_out, cnts_out,
               chunk_s, chunk_a, fs0, fa0, fs1, fa1, cv):
    wid = lax.axis_index("s") * 2 + lax.axis_index("c")
    lo0 = wid * (2 * BW)
    hi0 = lo0 + BW
    hi1 = lo0 + 2 * BW
    lanes = lax.iota(jnp.int32, 16)

    def chunk_body(ch, fills):
        pltpu.sync_copy(srcs_in.at[pl.ds(ch * CH, CH)], chunk_s)
        pltpu.sync_copy(aux_in.at[pl.ds(ch * CH, CH)], chunk_a)

        def vbody(g, fs):
            f0, f1 = fs
            a = chunk_a[pl.ds(g * 16, 16)]
            s = chunk_s[pl.ds(g * 16, 16)]
            d = lax.shift_right_logical(a, 16)
            m0 = jnp.logical_and(d >= lo0, d < hi0)
            m1 = jnp.logical_and(d >= hi0, d < hi1)
            c0 = plsc.cumsum(m0.astype(jnp.int32))
            c1 = plsc.cumsum(m1.astype(jnp.int32))
            trash = lanes + (CAP - 160)
            p0 = jnp.where(m0, f0 + c0 - 1, trash)
            p1 = jnp.where(m1, f1 + c1 - 1, trash)
            plsc.store_scatter(fs0, [p0], s)
            plsc.store_scatter(fa0, [p0], a)
            plsc.store_scatter(fs1, [p1], s)
            plsc.store_scatter(fa1, [p1], a)
            f0 = jnp.minimum(f0 + c0[15], CAP - 160)
            f1 = jnp.minimum(f1 + c1[15], CAP - 160)
            return (f0, f1)

        return lax.fori_loop(0, CH // 16, vbody, fills)

    f0, f1 = lax.fori_loop(0, EPAD // CH, chunk_body,
                           (jnp.int32(0), jnp.int32(0)))
    deada = jnp.full((16,), DEAD, jnp.int32)
    zeros = jnp.zeros((16,), jnp.int32)
    for kk in range(8):
        fa0[pl.ds(f0 + kk * 16, 16)] = deada
        fs0[pl.ds(f0 + kk * 16, 16)] = zeros
        fa1[pl.ds(f1 + kk * 16, 16)] = deada
        fs1[pl.ds(f1 + kk * 16, 16)] = zeros
    cp0 = jnp.maximum(lax.shift_left(lax.shift_right_logical(f0 + 127, 7), 7), 128)
    cp1 = jnp.maximum(lax.shift_left(lax.shift_right_logical(f1 + 127, 7), 7), 128)
    lanes = lax.iota(jnp.int32, 16)
    cvec = jnp.where(lanes == 0, cp0, jnp.where(lanes == 8, cp1, 0))
    cv[...] = cvec
    pltpu.sync_copy(cv, cnts_out.at[pl.ds(wid * 16, 16)])
    b0 = wid * 2
    pltpu.sync_copy(fs0, srcs_out.at[pl.ds(b0 * CAP, CAP)])
    pltpu.sync_copy(fa0, aux_out.at[pl.ds(b0 * CAP, CAP)])
    pltpu.sync_copy(fs1, srcs_out.at[pl.ds((b0 + 1) * CAP, CAP)])
    pltpu.sync_copy(fa1, aux_out.at[pl.ds((b0 + 1) * CAP, CAP)])


def _make_edge_kernel(sqrtc):
    @functools.partial(
        pl.kernel,
        out_type=jax.ShapeDtypeStruct((NB * ACC_BLK,), jnp.float32),
        mesh=_mesh,
        compiler_params=pltpu.CompilerParams(needs_layout_passes=False, use_tc_tiling_on_sc=False),
        scratch_types=[
            pltpu.VMEM((ACC_BLK,), jnp.float32),
            pltpu.VMEM((NPAD // NB * 20,), jnp.float32),
            pltpu.VMEM((BATCH, 48), jnp.float32),
            pltpu.VMEM((BATCH,), jnp.int32),
            pltpu.VMEM((BATCH,), jnp.int32),
            pltpu.VMEM((1024,), jnp.int32),
            pltpu.VMEM((2, 16), jnp.float32),
            pltpu.SemaphoreType.DMA,
        ],
    )
    def _edge(srcs_ref, aux_ref, cnts_ref, stab_ref, dtab_ref, ra_ref, out_ref,
              acc_v, dtab_v, rows_v, idx_v, auxb_v, cnt_v, ra_v, gsem):
        wid = lax.axis_index("s") * 2 + lax.axis_index("c")
        pltpu.sync_copy(cnts_ref, cnt_v)
        pltpu.sync_copy(ra_ref, ra_v)
        lanes = lax.iota(jnp.int32, 16)
        ra0 = ra_v[0, :]
        ra1 = ra_v[1, :]

        for t in range(2):
            b = wid * 2 + t
            nodebase = b * BW
            cntp = cnt_v[pl.ds(wid * 16 + t * 8, 16)][0]
            nb = lax.shift_right_logical(cntp, 7)
            ebase = b * CAP

            def zbody(i, _):
                acc_v[pl.ds(i * 16, 16)] = jnp.zeros((16,), jnp.float32)
                return 0
            lax.fori_loop(0, ACC_BLK // 16, zbody, 0)

            pltpu.sync_copy(dtab_ref.at[pl.ds(nodebase * 20, BW * 20)], dtab_v)

            def batch_body(bi, _):
                off = ebase + bi * BATCH
                pltpu.sync_copy(srcs_ref.at[pl.ds(off, BATCH)], idx_v)
                pltpu.sync_copy(aux_ref.at[pl.ds(off, BATCH)], auxb_v)
                pltpu.async_copy(stab_ref.at[idx_v], rows_v, gsem).wait()

                def group_body(g, _):
                    a = auxb_v[pl.ds(g * 16, 16)]
                    dstg = lax.shift_right_logical(a, 16)
                    ewf = lax.bitcast_convert_type(
                        lax.shift_left(a, 16), jnp.float32)
                    dlr = dstg - nodebase
                    validf = jnp.where(dlr < BW, 1.0, 0.0)
                    dl = jnp.clip(dlr, 0, BW - 1)
                    abase = dl * ACC_W
                    dbase = dl * 20
                    eid = lanes + g * 16
                    lg = jnp.zeros((16,), jnp.float32)
                    for j in range(16):
                        qj = plsc.load_gather(dtab_v, [dbase + j])
                        kj = plsc.load_gather(
                            rows_v, [eid, jnp.full((16,), j, jnp.int32)])
                        lg = lg + qj * kj
                    csh = plsc.load_gather(dtab_v, [dbase + 19])
                    el = jnp.exp(lg / sqrtc - csh) * validf
                    rel = []
                    for d in range(3):
                        xd = plsc.load_gather(dtab_v, [dbase + 16 + d])
                        xs = plsc.load_gather(
                            rows_v, [eid, jnp.full((16,), 16 + d, jnp.int32)])
                        rel.append(xd - xs)
                    rr = rel[0] * rel[0] + rel[1] * rel[1] + rel[2] * rel[2] + 1e-12
                    yb = jnp.int32(0x5F3759DF) - lax.shift_right_logical(
                        lax.bitcast_convert_type(rr, jnp.int32), 1)
                    y = lax.bitcast_convert_type(yb, jnp.float32)
                    for _i in range(3):
                        y = y * (1.5 - 0.5 * rr * y * y)
                    r = rr * y
                    dirs = [rel[0] * y, rel[1] * y, rel[2] * y]
                    rbf = _bf16r(r)
                    for tt in range(24):
                        wf = plsc.load_gather(
                            rows_v, [eid, jnp.full((16,), 24 + tt, jnp.int32)])
                        wi = lax.bitcast_convert_type(wf, jnp.int32)
                        h0 = lax.bitcast_convert_type(
                            lax.shift_left(wi, 16), jnp.float32)
                        h1 = lax.bitcast_convert_type(
                            jnp.bitwise_and(wi, M16), jnp.float32)
                        plsc.addupdate_scatter(acc_v, [abase + 2 * tt], el * h0)
                        plsc.addupdate_scatter(acc_v, [abase + 2 * tt + 1], el * h1)
                    for cc in range(16):
                        fc = jnp.maximum(rbf * ra0[cc] + ewf * ra1[cc], 0.0)
                        gf = el * _bf16r(fc)
                        for d in range(3):
                            plsc.addupdate_scatter(
                                acc_v, [abase + 48 + cc * 3 + d], gf * dirs[d])
                    plsc.addupdate_scatter(acc_v, [abase + 96], el)
                    return 0

                lax.fori_loop(0, BATCH // 16, group_body, 0)
                return 0

            lax.fori_loop(0, nb, batch_body, 0)
            pltpu.sync_copy(acc_v, out_ref.at[pl.ds(b * ACC_BLK, ACC_BLK)])

    return _edge


_edge_kernels = {}


def _edge_kernel(sqrtc):
    if sqrtc not in _edge_kernels:
        _edge_kernels[sqrtc] = _make_edge_kernel(sqrtc)
    return _edge_kernels[sqrtc]


def _prep(h, x, Wq, Wk, sqrtc):
    c = Wq.shape[1]
    inv = jnp.sqrt(jnp.sum(h * h, axis=-1) + 1e-12)
    q = inv @ Wq
    k = inv @ Wk
    q16 = jnp.pad(q, ((0, 0), (0, 16 - c)))
    k16 = jnp.pad(k, ((0, 0), (0, 16 - c)))
    kmax = jnp.max(jnp.sqrt(jnp.sum(k * k, axis=-1)))
    csh = jnp.sqrt(jnp.sum(q * q, axis=-1)) * kmax / sqrtc
    hb = h.reshape(N, 48).astype(jnp.bfloat16)
    hpk = lax.bitcast_convert_type(hb.reshape(N, 24, 2), jnp.float32)
    srctab = jnp.concatenate(
        [k16, x, jnp.zeros((N, 5), jnp.float32), hpk], axis=1)
    dsttab = jnp.concatenate([q16, x, csh[:, None]], axis=1)
    dsttab = jnp.concatenate(
        [dsttab, jnp.zeros((NPAD - N, 20), jnp.float32)], axis=0)
    return srctab, dsttab


def _finalize(acc, h, Wv, Rb, Ws, scale, bias, last):
    hs = acc[:, :48].reshape(N, 16, 3)
    fs = acc[:, 48:96].reshape(N, 16, 3)
    den = jnp.maximum(acc[:, 96], 1e-38)
    r32 = lambda w: w.astype(jnp.bfloat16).astype(jnp.float32)
    aggH = jnp.einsum('nmd,mc->ncd', hs, r32(Wv), precision='highest')
    aggF = jnp.einsum('njd,jc->ncd', fs, r32(Rb), precision='highest')
    self_ = jnp.einsum('nmd,mc->ncd', h, Ws)
    out = (aggH + aggF) / den[:, None, None] + self_
    if last:
        return out
    nrm = jnp.sqrt(jnp.sum(out * out, axis=-1) + 1e-12)
    return out / nrm[..., None] * jax.nn.relu(nrm * scale + bias)[..., None]


def kernel(x, v, edge_index, edge_w, Wq0, Wk0, Wv0, Ws0, Ra0, Rb0,
           Wq1, Wk1, Wv1, Ws1, Ra1, Rb1, Wq2, Wk2, Wv2, Ws2, Ra2, Rb2,
           scale0, bias0, scale1, bias1):
    src = edge_index[0]
    dst = edge_index[1]
    ewb = lax.bitcast_convert_type(
        edge_w[:, 0].astype(jnp.bfloat16), jnp.uint16).astype(jnp.int32)
    aux = jnp.bitwise_or(lax.shift_left(dst, 16), ewb)
    srcp = jnp.concatenate([src, jnp.zeros((EPAD - E,), jnp.int32)])
    auxp = jnp.concatenate([aux, jnp.full((EPAD - E,), DEAD, jnp.int32)])
    srcs_p, aux_p, cnts = _partition(srcp, auxp)

    layers = [
        (Wq0, Wk0, Wv0, Ws0, Ra0, Rb0, scale0, bias0, False),
        (Wq1, Wk1, Wv1, Ws1, Ra1, Rb1, scale1, bias1, False),
        (Wq2, Wk2, Wv2, Ws2, Ra2, Rb2, None, None, True),
    ]
    h = v
    for (Wq, Wk, Wv, Ws, Ra, Rb, scale, bias, last) in layers:
        sqrtc = float(math.sqrt(Wq.shape[1]))
        srctab, dsttab = _prep(h, x, Wq, Wk, sqrtc)
        ra_r = Ra.astype(jnp.bfloat16).astype(jnp.float32)
        accf = _edge_kernel(sqrtc)(srcs_p, aux_p, cnts, srctab,
                                   dsttab.reshape(-1), ra_r)
        acc = accf.reshape(NB, ACC_BLK)[:, :BW * ACC_W].reshape(NPAD, ACC_W)[:N]
        h = _finalize(acc, h, Wv, Rb, Ws, scale, bias, last)
    return h
